# baseline (speedup 1.0000x reference)
import functools

import jax
import jax.numpy as jnp
from jax import lax
from jax.experimental import pallas as pl
from jax.experimental.pallas import tpu as pltpu
from jax.experimental.pallas import tpu_sc as plsc

VOCAB = 1000000
EMBED_DIM = 16
BATCH = 16384

_info = plsc.get_sparse_core_info()
_NC, _NS = _info.num_cores, _info.num_subcores
_NW = _NC * _NS
_B_PER_W = BATCH // _NW
_LANES = 16
_WAVE = 8
_N_WAVES = _B_PER_W // _WAVE
_NBUF = 6

_mesh = plsc.VectorSubcoreMesh(core_axis_name="c", subcore_axis_name="s")


@functools.partial(
    pl.kernel,
    mesh=_mesh,
    out_type=jax.ShapeDtypeStruct((EMBED_DIM, BATCH), jnp.float32),
    scratch_types=[
        pltpu.VMEM((_B_PER_W,), jnp.int32),
        pltpu.VMEM((_NBUF, _WAVE, EMBED_DIM, 128), jnp.float32),
        pltpu.VMEM((EMBED_DIM, _B_PER_W), jnp.float32),
        pltpu.SemaphoreType.DMA((_NBUF,)),
    ],
    compiler_params=pltpu.CompilerParams(needs_layout_passes=False),
)
def _gather_kernel(table_hbm, idx_hbm, out_hbm, idx_v, blocks_v, stage_v,
                   sems):
    wid = lax.axis_index("s") * _NC + lax.axis_index("c")
    base = wid * _B_PER_W
    pltpu.sync_copy(idx_hbm.at[pl.ds(base, _B_PER_W)], idx_v)

    lanes = lax.iota(jnp.int32, _LANES)
    zero128 = pl.multiple_of(jnp.int32(0), 128)

    def issue_wave(k, buf):
        vec = idx_v[pl.ds((k // 2) * _LANES, _LANES)]
        vbase_vec = (vec >> 7) << 7
        lbase = (k % 2) * _WAVE
        for l in range(_WAVE):
            vbase = jnp.sum(jnp.where(lanes == lbase + l, vbase_vec, 0))
            vbase = pl.multiple_of(vbase, 128)
            pltpu.async_copy(
                table_hbm.at[lanes, pl.ds(vbase, 128)],
                blocks_v.at[buf, l],
                sems.at[buf],
            )

    def drain_wave(buf):
        for l in range(_WAVE):
            pltpu.make_async_copy(
                table_hbm.at[lanes, pl.ds(zero128, 128)],
                blocks_v.at[buf, l],
                sems.at[buf],
            ).wait()

    def select_wave(k, buf):
        vec = idx_v[pl.ds((k // 2) * _LANES, _LANES)]
        cvec = vec & 127
        bufvec = jnp.full((_LANES,), buf, jnp.int32)
        lbase = (k % 2) * _WAVE
        for l in range(_WAVE):
            c = jnp.sum(jnp.where(lanes == lbase + l, cvec, 0))
            vals = plsc.load_gather(
                blocks_v,
                [
                    bufvec,
                    jnp.full((_LANES,), l, jnp.int32),
                    lanes,
                    jnp.full((_LANES,), c, jnp.int32),
                ],
            )
            plsc.store_scatter(
                stage_v,
                [lanes, jnp.full((_LANES,), k * _WAVE + l, jnp.int32)],
                vals,
            )

    for p in range(_NBUF - 1):
        issue_wave(p, jnp.int32(p))

    def wave_body(k, carry):
        buf = lax.rem(k, _NBUF)
        nxt = lax.rem(k + _NBUF - 1, _NBUF)

        @pl.when(k + _NBUF - 1 < _N_WAVES)
        def _():
            issue_wave(k + _NBUF - 1, nxt)

        drain_wave(buf)
        select_wave(k, buf)
        return carry

    lax.fori_loop(0, _N_WAVES, wave_body, 0)

    pltpu.sync_copy(stage_v, out_hbm.at[:, pl.ds(base, _B_PER_W)])


def kernel(g, x, attri):
    idx = jnp.squeeze(x).astype(jnp.int32)
    out_t = _gather_kernel(attri.T, idx)
    return out_t.T


# speedup vs baseline: 1.0286x; 1.0286x over previous
import functools

import jax
import jax.numpy as jnp
from jax import lax
from jax.experimental import pallas as pl
from jax.experimental.pallas import tpu as pltpu
from jax.experimental.pallas import tpu_sc as plsc

VOCAB = 1000000
EMBED_DIM = 16
BATCH = 16384

_info = plsc.get_sparse_core_info()
_NC, _NS = _info.num_cores, _info.num_subcores
_NW = _NC * _NS
_B_PER_W = BATCH // _NW
_LANES = 16
_WAVE = 16
_N_WAVES = _B_PER_W // _WAVE
_NBUF = 3

_mesh = plsc.VectorSubcoreMesh(core_axis_name="c", subcore_axis_name="s")


@functools.partial(
    pl.kernel,
    mesh=_mesh,
    out_type=jax.ShapeDtypeStruct((EMBED_DIM, BATCH), jnp.float32),
    scratch_types=[
        pltpu.VMEM((_B_PER_W,), jnp.int32),
        pltpu.VMEM((_NBUF, _WAVE, EMBED_DIM, 128), jnp.float32),
        pltpu.VMEM((EMBED_DIM, _B_PER_W), jnp.float32),
        pltpu.SemaphoreType.DMA((_NBUF,)),
    ],
    compiler_params=pltpu.CompilerParams(needs_layout_passes=False),
)
def _gather_kernel(table_hbm, idx_hbm, out_hbm, idx_v, blocks_v, stage_v,
                   sems):
    wid = lax.axis_index("s") * _NC + lax.axis_index("c")
    base = wid * _B_PER_W
    pltpu.sync_copy(idx_hbm.at[pl.ds(base, _B_PER_W)], idx_v)

    lanes = lax.iota(jnp.int32, _LANES)
    zero128 = pl.multiple_of(jnp.int32(0), 128)

    def issue_wave(k, buf):
        vec = idx_v[pl.ds(k * _WAVE, _WAVE)]
        vbase_vec = (vec >> 7) << 7
        for l in range(_WAVE):
            vbase = vbase_vec[l]
            vbase = pl.multiple_of(vbase, 128)
            pltpu.async_copy(
                table_hbm.at[lanes, pl.ds(vbase, 128)],
                blocks_v.at[buf, l],
                sems.at[buf],
            )

    def drain_wave(buf):
        for l in range(_WAVE):
            pltpu.make_async_copy(
                table_hbm.at[lanes, pl.ds(zero128, 128)],
                blocks_v.at[buf, l],
                sems.at[buf],
            ).wait()

    def select_wave(k, buf):
        vec = idx_v[pl.ds(k * _WAVE, _WAVE)]
        cvec = vec & 127
        bufvec = jnp.full((_LANES,), buf, jnp.int32)
        for l in range(_WAVE):
            c = cvec[l]
            vals = plsc.load_gather(
                blocks_v,
                [
                    bufvec,
                    jnp.full((_LANES,), l, jnp.int32),
                    lanes,
                    jnp.full((_LANES,), c, jnp.int32),
                ],
            )
            plsc.store_scatter(
                stage_v,
                [lanes, jnp.full((_LANES,), k * _WAVE + l, jnp.int32)],
                vals,
            )

    for p in range(_NBUF - 1):
        issue_wave(p, jnp.int32(p))

    def wave_body(k, carry):
        buf = lax.rem(k, _NBUF)
        nxt = lax.rem(k + _NBUF - 1, _NBUF)

        @pl.when(k + _NBUF - 1 < _N_WAVES)
        def _():
            issue_wave(k + _NBUF - 1, nxt)

        drain_wave(buf)
        select_wave(k, buf)
        return carry

    lax.fori_loop(0, _N_WAVES, wave_body, 0)

    pltpu.sync_copy(stage_v, out_hbm.at[:, pl.ds(base, _B_PER_W)])


def kernel(g, x, attri):
    idx = jnp.squeeze(x).astype(jnp.int32)
    out_t = _gather_kernel(attri.T, idx)
    return out_t.T
